# baseline (speedup 1.0000x reference)
import jax
import jax.numpy as jnp
from jax.experimental import pallas as pl
from jax.experimental.pallas import tpu as pltpu

_SB = 32
_NBUF = 4


def kernel(x, pe_weight):
    batch = x.shape[0]
    max_len, d_model = pe_weight.shape
    sb = _SB if batch % _SB == 0 else 1
    n_copies = batch // sb

    def body(pe_ref, out_ref, scratch_ref, sem):
        scratch_ref[...] = jnp.broadcast_to(
            pe_ref[...][None, None, :, :], scratch_ref.shape
        )
        copies = [
            pltpu.make_async_copy(
                scratch_ref.at[i % _NBUF],
                out_ref.at[pl.ds(i * sb, sb)],
                sem,
            )
            for i in range(n_copies)
        ]
        for c in copies:
            c.start()
        for c in copies:
            c.wait()

    return pl.pallas_call(
        body,
        in_specs=[pl.BlockSpec(memory_space=pltpu.MemorySpace.VMEM)],
        out_specs=pl.BlockSpec(memory_space=pl.ANY),
        out_shape=jax.ShapeDtypeStruct((batch, max_len, d_model), pe_weight.dtype),
        scratch_shapes=[
            pltpu.VMEM((_NBUF, sb, max_len, d_model), pe_weight.dtype),
            pltpu.SemaphoreType.DMA,
        ],
    )(pe_weight)


# speedup vs baseline: 1.0855x; 1.0855x over previous
import jax
import jax.numpy as jnp
from jax.experimental import pallas as pl
from jax.experimental.pallas import tpu as pltpu

_BB = 32


def _bcast_body(pe_ref, out_ref):
    @pl.when(pl.program_id(0) < 2)
    def _fill():
        out_ref[...] = jnp.broadcast_to(pe_ref[...][None, :, :], out_ref.shape)


def kernel(x, pe_weight):
    batch = x.shape[0]
    max_len, d_model = pe_weight.shape
    bb = _BB if batch % _BB == 0 else 1
    return pl.pallas_call(
        _bcast_body,
        grid=(batch // bb,),
        in_specs=[pl.BlockSpec(memory_space=pltpu.MemorySpace.VMEM)],
        out_specs=pl.BlockSpec((bb, max_len, d_model), lambda i: (i, 0, 0)),
        out_shape=jax.ShapeDtypeStruct((batch, max_len, d_model), pe_weight.dtype),
    )(pe_weight)
